# baseline (speedup 1.0000x reference)
import jax
import jax.numpy as jnp
from jax import lax
from jax.experimental import pallas as pl
from jax.experimental.pallas import tpu as pltpu
from jax.experimental.pallas import tpu_sc as plsc

_NW = 32
_N = 8 * 64 * 64 * 384
_PW = _N // _NW
_C = 6144
_NCH = _PW // _C
_VPC = _C // 16


def _sc_body(x1, x2, x3, o1, o2, o3,
             xa0, xb0, xc0, ya0, yb0, yc0,
             xa1, xb1, xc1, ya1, yb1, yc1,
             gs0, gs1, ss0, ss1):
    wid = lax.axis_index("s") * 2 + lax.axis_index("c")
    base = wid * _PW
    lane = lax.iota(jnp.int32, 16) & 3
    m0 = lane == 0
    m2 = lane == 2

    xs = ((xa0, xb0, xc0), (xa1, xb1, xc1))
    ys = ((ya0, yb0, yc0), (ya1, yb1, yc1))
    gsems = (gs0, gs1)
    ssems = (ss0, ss1)
    srcs = (x1, x2, x3)
    outs = (o1, o2, o3)

    def compute(par):
        xa, xb, xc = xs[par]
        ya, yb, yc = ys[par]

        def body(j, _):
            s = pl.ds(j * 16, 16)
            a = xa[s]
            b = xb[s]
            c = xc[s]
            ya[s] = jnp.where(m0, c, jnp.where(m2, b, a))
            yb[s] = jnp.where(m0, c, jnp.where(m2, a, b))
            yc[s] = jnp.where(m0, b, jnp.where(m2, a, c))
            return 0

        lax.fori_loop(0, _VPC, body, 0)

    gpend = [None, None]
    spend = [None, None]
    for i in range(_NCH + 1):
        par = i % 2
        if i < _NCH:
            if spend[par] is not None:
                for h in spend[par]:
                    h.wait()
                spend[par] = None
            sl = pl.ds(base + i * _C, _C)
            hs = []
            for k in range(3):
                g = pltpu.make_async_copy(srcs[k].at[sl], xs[par][k],
                                          gsems[par])
                g.start()
                hs.append(g)
            gpend[par] = hs
        if i >= 1:
            q = 1 - par
            for h in gpend[q]:
                h.wait()
            gpend[q] = None
            compute(q)
            sl = pl.ds(base + (i - 1) * _C, _C)
            hs = []
            for k in range(3):
                s = pltpu.make_async_copy(ys[q][k], outs[k].at[sl], ssems[q])
                s.start()
                hs.append(s)
            spend[q] = hs
    for hs in spend:
        if hs is not None:
            for h in hs:
                h.wait()


def kernel(x1, x2, x3):
    b, ch, h, w = x1.shape
    flat = lambda x: x.transpose(0, 2, 3, 1).reshape(_N)
    mesh = plsc.VectorSubcoreMesh(core_axis_name="c", subcore_axis_name="s")
    vbuf = pltpu.VMEM((_C,), jnp.float32)
    f = pl.kernel(
        _sc_body,
        mesh=mesh,
        out_type=[jax.ShapeDtypeStruct((_N,), jnp.float32)] * 3,
        scratch_types=[vbuf] * 12 + [pltpu.SemaphoreType.DMA] * 4,
    )
    y1, y2, y3 = f(flat(x1), flat(x2), flat(x3))
    unflat = lambda y: y.reshape(b, h, w, ch).transpose(0, 3, 1, 2)
    return (unflat(y1), unflat(y2), unflat(y3))


# speedup vs baseline: 4.8816x; 4.8816x over previous
import jax
import jax.numpy as jnp
from jax.experimental import pallas as pl

_HB = 32


def _exchange_kernel(x1_ref, x2_ref, x3_ref, o1_ref, o2_ref, o3_ref):
    m = jax.lax.broadcasted_iota(jnp.int32, x1_ref.shape, 3) & 3
    m0 = m == 0
    m2 = m == 2
    a = x1_ref[...]
    b = x2_ref[...]
    c = x3_ref[...]
    o1_ref[...] = jnp.where(m0, c, jnp.where(m2, b, a))
    o2_ref[...] = jnp.where(m0, c, jnp.where(m2, a, b))
    o3_ref[...] = jnp.where(m0, b, jnp.where(m2, a, c))


def kernel(x1, x2, x3):
    b, ch, h, w = x1.shape
    t = lambda x: x.transpose(0, 2, 3, 1)
    spec = pl.BlockSpec((1, _HB, w, ch), lambda i, j: (i, j, 0, 0))
    y1, y2, y3 = pl.pallas_call(
        _exchange_kernel,
        grid=(b, h // _HB),
        in_specs=[spec, spec, spec],
        out_specs=[spec, spec, spec],
        out_shape=[jax.ShapeDtypeStruct((b, h, w, ch), x1.dtype)] * 3,
    )(t(x1), t(x2), t(x3))
    u = lambda y: y.transpose(0, 3, 1, 2)
    return (u(y1), u(y2), u(y3))
